# baseline (speedup 1.0000x reference)
import functools

import jax
import jax.numpy as jnp
from jax import lax
from jax.experimental import pallas as pl
from jax.experimental.pallas import tpu as pltpu
from jax.experimental.pallas import tpu_sc as plsc

N = 10000
D = 128
E = 320000

NC = 2
NS = 16
NW = NC * NS
C = 128
EW = 10240
EP = NW * EW
NH = 5000
NPH = 5120
TRASH = NPH - 1
SCAP = 85
NBUF = 4
RPT = NPH // NS
NCNT = NC * NH + 240


def _partition_body(src_hbm, dst_hbm, seg_src, seg_dst, lens_out, cnt_out,
                    sv, dv, ls, ld, hs, hd, cnt_loc, lens_buf):
    c = lax.axis_index("c")
    s = lax.axis_index("s")
    w = c * NS + s

    pltpu.sync_copy(src_hbm.at[w], sv)
    pltpu.sync_copy(dst_hbm.at[w], dv)

    zf = jnp.zeros((16,), jnp.float32)
    zi = jnp.zeros((16,), jnp.int32)
    ti = jnp.full((16,), TRASH, jnp.int32)

    def pre(i, carry):
        o = 16 * i
        ls[pl.ds(o, 16)] = zi
        ld[pl.ds(o, 16)] = ti
        hs[pl.ds(o, 16)] = zi
        hd[pl.ds(o, 16)] = ti
        return carry

    lax.fori_loop(0, SCAP * C // 16, pre, 0)

    def zc(i, carry):
        cnt_loc[pl.ds(16 * i, 16)] = zf
        return carry

    lax.fori_loop(0, NCNT // 16, zc, 0)

    ones = jnp.ones((16,), jnp.float32)
    iota = lax.iota(jnp.int32, 16)
    base0 = w * EW

    def step(i, carry):
        nlow, nhigh = carry
        sv16 = sv[pl.ds(16 * i, 16)]
        dv16 = dv[pl.ds(16 * i, 16)]
        sv16 = jnp.where(sv16 >= NH, sv16 + (NPH - NH), sv16)
        valid = (base0 + 16 * i + iota) < E
        mlow = jnp.logical_and(dv16 < NH, valid)
        mhigh = jnp.logical_and(dv16 >= NH, valid)
        plsc.addupdate_scatter(cnt_loc, [dv16], ones, mask=valid)
        plsc.store_compressed(ls.at[pl.ds(nlow, 16)], sv16, mask=mlow)
        plsc.store_compressed(ld.at[pl.ds(nlow, 16)], dv16, mask=mlow)
        plsc.store_compressed(hs.at[pl.ds(nhigh, 16)], sv16, mask=mhigh)
        plsc.store_compressed(hd.at[pl.ds(nhigh, 16)], dv16 - NH, mask=mhigh)
        nlow = nlow + jnp.max(plsc.all_reduce_population_count(mlow))
        nhigh = nhigh + jnp.max(plsc.all_reduce_population_count(mhigh))
        return nlow, nhigh

    nlow, nhigh = lax.fori_loop(0, EW // 16, step,
                                (jnp.int32(0), jnp.int32(0)))

    ntl = jnp.maximum((nlow + NBUF * C - 1) // (NBUF * C), 1) * NBUF
    nth = jnp.maximum((nhigh + NBUF * C - 1) // (NBUF * C), 1) * NBUF
    lens_buf[pl.ds(0, 16)] = jnp.where(iota == 0, ntl,
                                       jnp.where(iota == 1, nth, 0))

    pltpu.sync_copy(ls, seg_src.at[w, 0])
    pltpu.sync_copy(ld, seg_dst.at[w, 0])
    pltpu.sync_copy(hs, seg_src.at[w, 1])
    pltpu.sync_copy(hd, seg_dst.at[w, 1])
    pltpu.sync_copy(lens_buf, lens_out.at[w])
    pltpu.sync_copy(cnt_loc, cnt_out.at[w])


def _make_partition():
    mesh = plsc.VectorSubcoreMesh(core_axis_name="c", subcore_axis_name="s")
    out_type = (
        jax.ShapeDtypeStruct((NW, NC, SCAP * C), jnp.int32),
        jax.ShapeDtypeStruct((NW, NC, SCAP * C), jnp.int32),
        jax.ShapeDtypeStruct((NW, 16), jnp.int32),
        jax.ShapeDtypeStruct((NW, NCNT), jnp.float32),
    )
    scratch = (
        pltpu.VMEM((EW,), jnp.int32),
        pltpu.VMEM((EW,), jnp.int32),
        pltpu.VMEM((SCAP * C,), jnp.int32),
        pltpu.VMEM((SCAP * C,), jnp.int32),
        pltpu.VMEM((SCAP * C,), jnp.int32),
        pltpu.VMEM((SCAP * C,), jnp.int32),
        pltpu.VMEM((NCNT,), jnp.float32),
        pltpu.VMEM((16,), jnp.int32),
    )
    return pl.kernel(
        _partition_body,
        out_type=out_type,
        mesh=mesh,
        scratch_types=scratch,
        compiler_params=pltpu.CompilerParams(use_tc_tiling_on_sc=False,
                                            needs_layout_passes=False),
    )


def _agg_body(x_hbm, seg_src, seg_dst, lens_hbm, acc_out,
              sv2, dv2, rows, lens_v, acc_sh, gsem, ssem):
    c = lax.axis_index("c")
    s = lax.axis_index("s")

    pltpu.sync_copy(lens_hbm, lens_v)

    zv = jnp.zeros((16,), jnp.float32)

    def zrow(r, carry):
        for q in range(D // 16):
            rows[0, r, pl.ds(16 * q, 16)] = zv
        return carry

    lax.fori_loop(0, C, zrow, 0)
    for t in range(RPT // C):
        pltpu.sync_copy(rows.at[0], acc_sh.at[pl.ds(s * RPT + t * C, C)])
    rem = RPT - (RPT // C) * C
    if rem:
        pltpu.sync_copy(rows.at[0].at[pl.ds(0, rem)],
                        acc_sh.at[pl.ds(s * RPT + (RPT // C) * C, rem)])

    plsc.subcore_barrier()

    for seg in range(2):
        w = 2 * s + seg
        pltpu.sync_copy(seg_src.at[w, c], sv2)
        pltpu.sync_copy(seg_dst.at[w, c], dv2)
        lv = lens_v[w]
        nt = jnp.where(c == 0, lv[0], lv[1])
        ntb = nt // NBUF

        for b in range(NBUF):
            pltpu.async_copy(x_hbm.at[sv2.at[b]], rows.at[b], gsem.at[b])

        def step(t2, carry):
            j0 = NBUF * t2
            for b in range(NBUF):
                j = j0 + b
                pltpu.make_async_copy(x_hbm.at[sv2.at[j]], rows.at[b],
                                      gsem.at[b]).wait()
                pltpu.async_copy(rows.at[b], acc_sh.at[dv2.at[j]],
                                 ssem.at[b], add=True)

            @pl.when(t2 < ntb - 1)
            def _():
                for b in range(NBUF):
                    j = j0 + b
                    pltpu.make_async_copy(rows.at[b], acc_sh.at[dv2.at[j]],
                                          ssem.at[b]).wait()
                    pltpu.async_copy(x_hbm.at[sv2.at[j + NBUF]], rows.at[b],
                                     gsem.at[b])

            return carry

        lax.fori_loop(0, ntb, step, 0)

        for b in range(NBUF):
            pltpu.make_async_copy(rows.at[b], acc_sh.at[dv2.at[b]],
                                  ssem.at[b]).wait()

    plsc.subcore_barrier()

    pltpu.sync_copy(acc_sh.at[pl.ds(s * RPT, RPT)],
                    acc_out.at[c, pl.ds(s * RPT, RPT)])


def _make_agg():
    mesh = plsc.VectorSubcoreMesh(core_axis_name="c", subcore_axis_name="s")
    out_type = jax.ShapeDtypeStruct((NC, NPH, D), jnp.float32)
    scratch = (
        pltpu.VMEM((SCAP, C), jnp.int32),
        pltpu.VMEM((SCAP, C), jnp.int32),
        pltpu.VMEM((NBUF, C, D), jnp.float32),
        pltpu.VMEM((NW, 16), jnp.int32),
        pltpu.VMEM_SHARED((NPH, D), jnp.float32),
        pltpu.SemaphoreType.DMA((NBUF,)),
        pltpu.SemaphoreType.DMA((NBUF,)),
    )
    return pl.kernel(
        _agg_body,
        out_type=out_type,
        mesh=mesh,
        scratch_types=scratch,
        compiler_params=pltpu.CompilerParams(use_tc_tiling_on_sc=False),
    )


def _combine_body(relu, pa_ref, pc_ref, xin_ref, wl_ref, wr_ref, b_ref,
                  out_ref):
    acc = pa_ref[0]
    cnt = jnp.sum(pc_ref[...], axis=1, keepdims=True)
    mean = acc * (1.0 / jnp.maximum(cnt, 1.0))
    y = (jnp.dot(mean, wl_ref[...], preferred_element_type=jnp.float32)
         + b_ref[...]
         + jnp.dot(xin_ref[...], wr_ref[...],
                   preferred_element_type=jnp.float32))
    out_ref[...] = jnp.maximum(y, 0.0) if relu else y


def _combine(pa, pcT, xin, wlT, wrT, b2d, relu):
    R = 1024
    grid = (NC * NPH // R,)
    return pl.pallas_call(
        functools.partial(_combine_body, relu),
        grid=grid,
        in_specs=[
            pl.BlockSpec((1, R, D), lambda i: (i // 5, i % 5, 0)),
            pl.BlockSpec((R, NW), lambda i: (i, 0)),
            pl.BlockSpec((R, D), lambda i: (i, 0)),
            pl.BlockSpec((D, D), lambda i: (0, 0)),
            pl.BlockSpec((D, D), lambda i: (0, 0)),
            pl.BlockSpec((1, D), lambda i: (0, 0)),
        ],
        out_specs=pl.BlockSpec((R, D), lambda i: (i, 0)),
        out_shape=jax.ShapeDtypeStruct((NC * NPH, D), jnp.float32),
    )(pa, pcT, xin, wlT, wrT, b2d)


@jax.jit
def kernel(x, edge_index, W1l, b1l, W1r, W2l, b2l, W2r):
    src = edge_index[0]
    dst = edge_index[1]
    pad = EP - E
    src_pp = jnp.concatenate([src, jnp.zeros((pad,), jnp.int32)]).reshape(NW, EW)
    dst_pp = jnp.concatenate([dst, jnp.zeros((pad,), jnp.int32)]).reshape(NW, EW)

    xp = jnp.zeros((NC * NPH, D), jnp.float32)
    xp = xp.at[:NH].set(x[:NH])
    xp = xp.at[NPH:NPH + NH].set(x[NH:])

    seg_src, seg_dst, lens, pcnt = _make_partition()(src_pp, dst_pp)
    seg_src = seg_src.reshape(NW, NC, SCAP, C)
    seg_dst = seg_dst.reshape(NW, NC, SCAP, C)
    pcT = jnp.zeros((NC * NPH, NW), jnp.float32)
    pcT = pcT.at[:NH].set(pcnt.T[:NH])
    pcT = pcT.at[NPH:NPH + NH].set(pcnt.T[NH:NC * NH])

    pa1 = _make_agg()(xp, seg_src, seg_dst, lens)
    h = _combine(pa1, pcT, xp, W1l.T, W1r.T, b1l.reshape(1, D), relu=True)
    pa2 = _make_agg()(h, seg_src, seg_dst, lens)
    outp = _combine(pa2, pcT, h, W2l.T, W2r.T, b2l.reshape(1, D), relu=False)
    return jnp.concatenate([outp[:NH], outp[NPH:NPH + NH]])


# speedup vs baseline: 1.7006x; 1.7006x over previous
import functools

import jax
import jax.numpy as jnp
from jax import lax
from jax.experimental import pallas as pl
from jax.experimental.pallas import tpu as pltpu
from jax.experimental.pallas import tpu_sc as plsc

N = 10000
D = 128
E = 320000

NC = 2
NS = 16
NW = NC * NS
C = 64
EW = 10240
EP = NW * EW
NH = 5000
NPH = 5120
TRASH = NPH - 1
SCAP = 170
NBUF = 4
RPT = NPH // NS
NCNT = NC * NH + 240


def _partition_body(src_hbm, dst_hbm, seg_src, seg_dst, lens_out, cnt_out,
                    sv, dv, ls, ld, hs, hd, cnt_loc, lens_buf):
    c = lax.axis_index("c")
    s = lax.axis_index("s")
    w = c * NS + s

    pltpu.sync_copy(src_hbm.at[w], sv)
    pltpu.sync_copy(dst_hbm.at[w], dv)

    zf = jnp.zeros((16,), jnp.float32)
    zi = jnp.zeros((16,), jnp.int32)
    ti = jnp.full((16,), TRASH, jnp.int32)

    def pre(i, carry):
        o = 16 * i
        ls[pl.ds(o, 16)] = zi
        ld[pl.ds(o, 16)] = ti
        hs[pl.ds(o, 16)] = zi
        hd[pl.ds(o, 16)] = ti
        return carry

    lax.fori_loop(0, SCAP * C // 16, pre, 0)

    def zc(i, carry):
        cnt_loc[pl.ds(16 * i, 16)] = zf
        return carry

    lax.fori_loop(0, NCNT // 16, zc, 0)

    ones = jnp.ones((16,), jnp.float32)
    iota = lax.iota(jnp.int32, 16)
    base0 = w * EW

    def step(i, carry):
        nlow, nhigh = carry
        sv16 = sv[pl.ds(16 * i, 16)]
        dv16 = dv[pl.ds(16 * i, 16)]
        sv16 = jnp.where(sv16 >= NH, sv16 + (NPH - NH), sv16)
        valid = (base0 + 16 * i + iota) < E
        mlow = jnp.logical_and(dv16 < NH, valid)
        mhigh = jnp.logical_and(dv16 >= NH, valid)
        plsc.addupdate_scatter(cnt_loc, [dv16], ones, mask=valid)
        plsc.store_compressed(ls.at[pl.ds(nlow, 16)], sv16, mask=mlow)
        plsc.store_compressed(ld.at[pl.ds(nlow, 16)], dv16, mask=mlow)
        plsc.store_compressed(hs.at[pl.ds(nhigh, 16)], sv16, mask=mhigh)
        plsc.store_compressed(hd.at[pl.ds(nhigh, 16)], dv16 - NH, mask=mhigh)
        nlow = nlow + jnp.max(plsc.all_reduce_population_count(mlow))
        nhigh = nhigh + jnp.max(plsc.all_reduce_population_count(mhigh))
        return nlow, nhigh

    nlow, nhigh = lax.fori_loop(0, EW // 16, step,
                                (jnp.int32(0), jnp.int32(0)))

    ntl = jnp.maximum((nlow + NBUF * C - 1) // (NBUF * C), 1) * NBUF
    nth = jnp.maximum((nhigh + NBUF * C - 1) // (NBUF * C), 1) * NBUF
    lens_buf[pl.ds(0, 16)] = jnp.where(iota == 0, ntl,
                                       jnp.where(iota == 1, nth, 0))

    pltpu.sync_copy(ls, seg_src.at[w, 0])
    pltpu.sync_copy(ld, seg_dst.at[w, 0])
    pltpu.sync_copy(hs, seg_src.at[w, 1])
    pltpu.sync_copy(hd, seg_dst.at[w, 1])
    pltpu.sync_copy(lens_buf, lens_out.at[w])
    pltpu.sync_copy(cnt_loc, cnt_out.at[w])


def _make_partition():
    mesh = plsc.VectorSubcoreMesh(core_axis_name="c", subcore_axis_name="s")
    out_type = (
        jax.ShapeDtypeStruct((NW, NC, SCAP * C), jnp.int32),
        jax.ShapeDtypeStruct((NW, NC, SCAP * C), jnp.int32),
        jax.ShapeDtypeStruct((NW, 16), jnp.int32),
        jax.ShapeDtypeStruct((NW, NCNT), jnp.float32),
    )
    scratch = (
        pltpu.VMEM((EW,), jnp.int32),
        pltpu.VMEM((EW,), jnp.int32),
        pltpu.VMEM((SCAP * C,), jnp.int32),
        pltpu.VMEM((SCAP * C,), jnp.int32),
        pltpu.VMEM((SCAP * C,), jnp.int32),
        pltpu.VMEM((SCAP * C,), jnp.int32),
        pltpu.VMEM((NCNT,), jnp.float32),
        pltpu.VMEM((16,), jnp.int32),
    )
    return pl.kernel(
        _partition_body,
        out_type=out_type,
        mesh=mesh,
        scratch_types=scratch,
        compiler_params=pltpu.CompilerParams(use_tc_tiling_on_sc=False,
                                            needs_layout_passes=False),
    )


def _agg_body(x_hbm, seg_src, seg_dst, lens_hbm, acc_out,
              sv2, dv2, rows, lens_v, acc_sh, gsem, ssem):
    c = lax.axis_index("c")
    s = lax.axis_index("s")

    pltpu.sync_copy(lens_hbm, lens_v)

    zv = jnp.zeros((16,), jnp.float32)

    def zrow(r, carry):
        for q in range(D // 16):
            rows[0, r, pl.ds(16 * q, 16)] = zv
        return carry

    lax.fori_loop(0, C, zrow, 0)
    for t in range(RPT // C):
        pltpu.sync_copy(rows.at[0], acc_sh.at[pl.ds(s * RPT + t * C, C)])
    rem = RPT - (RPT // C) * C
    if rem:
        pltpu.sync_copy(rows.at[0].at[pl.ds(0, rem)],
                        acc_sh.at[pl.ds(s * RPT + (RPT // C) * C, rem)])

    plsc.subcore_barrier()

    for seg in range(2):
        w = 2 * s + seg
        pltpu.sync_copy(seg_src.at[w, c], sv2)
        pltpu.sync_copy(seg_dst.at[w, c], dv2)
        lv = lens_v[w]
        nt = jnp.where(c == 0, lv[0], lv[1])
        ntb = nt // NBUF

        for b in range(NBUF):
            pltpu.async_copy(x_hbm.at[sv2.at[b]], rows.at[b], gsem.at[b])

        def step(t2, carry):
            j0 = NBUF * t2
            for b in range(NBUF):
                j = j0 + b
                pltpu.make_async_copy(x_hbm.at[sv2.at[j]], rows.at[b],
                                      gsem.at[b]).wait()
                pltpu.async_copy(rows.at[b], acc_sh.at[dv2.at[j]],
                                 ssem.at[b], add=True)

            @pl.when(t2 < ntb - 1)
            def _():
                for b in range(NBUF):
                    j = j0 + b
                    pltpu.make_async_copy(rows.at[b], acc_sh.at[dv2.at[j]],
                                          ssem.at[b]).wait()
                    pltpu.async_copy(x_hbm.at[sv2.at[j + NBUF]], rows.at[b],
                                     gsem.at[b])

            return carry

        lax.fori_loop(0, ntb, step, 0)

        for b in range(NBUF):
            pltpu.make_async_copy(rows.at[b], acc_sh.at[dv2.at[b]],
                                  ssem.at[b]).wait()

    plsc.subcore_barrier()

    pltpu.sync_copy(acc_sh.at[pl.ds(s * RPT, RPT)],
                    acc_out.at[c, pl.ds(s * RPT, RPT)])


def _make_agg():
    mesh = plsc.VectorSubcoreMesh(core_axis_name="c", subcore_axis_name="s")
    out_type = jax.ShapeDtypeStruct((NC, NPH, D), jnp.float32)
    scratch = (
        pltpu.VMEM((SCAP, C), jnp.int32),
        pltpu.VMEM((SCAP, C), jnp.int32),
        pltpu.VMEM((NBUF, C, D), jnp.float32),
        pltpu.VMEM((NW, 16), jnp.int32),
        pltpu.VMEM_SHARED((NPH, D), jnp.float32),
        pltpu.SemaphoreType.DMA((NBUF,)),
        pltpu.SemaphoreType.DMA((NBUF,)),
    )
    return pl.kernel(
        _agg_body,
        out_type=out_type,
        mesh=mesh,
        scratch_types=scratch,
        compiler_params=pltpu.CompilerParams(use_tc_tiling_on_sc=False),
    )


def _combine_body(relu, pa_ref, pc_ref, xin_ref, wl_ref, wr_ref, b_ref,
                  out_ref):
    acc = pa_ref[0]
    cnt = jnp.sum(pc_ref[...], axis=1, keepdims=True)
    mean = acc * (1.0 / jnp.maximum(cnt, 1.0))
    y = (jnp.dot(mean, wl_ref[...], preferred_element_type=jnp.float32)
         + b_ref[...]
         + jnp.dot(xin_ref[...], wr_ref[...],
                   preferred_element_type=jnp.float32))
    out_ref[...] = jnp.maximum(y, 0.0) if relu else y


def _combine(pa, pcT, xin, wlT, wrT, b2d, relu):
    R = 1024
    grid = (NC * NPH // R,)
    return pl.pallas_call(
        functools.partial(_combine_body, relu),
        grid=grid,
        in_specs=[
            pl.BlockSpec((1, R, D), lambda i: (i // 5, i % 5, 0)),
            pl.BlockSpec((R, NW), lambda i: (i, 0)),
            pl.BlockSpec((R, D), lambda i: (i, 0)),
            pl.BlockSpec((D, D), lambda i: (0, 0)),
            pl.BlockSpec((D, D), lambda i: (0, 0)),
            pl.BlockSpec((1, D), lambda i: (0, 0)),
        ],
        out_specs=pl.BlockSpec((R, D), lambda i: (i, 0)),
        out_shape=jax.ShapeDtypeStruct((NC * NPH, D), jnp.float32),
    )(pa, pcT, xin, wlT, wrT, b2d)


@jax.jit
def kernel(x, edge_index, W1l, b1l, W1r, W2l, b2l, W2r):
    src = edge_index[0]
    dst = edge_index[1]
    pad = EP - E
    src_pp = jnp.concatenate([src, jnp.zeros((pad,), jnp.int32)]).reshape(NW, EW)
    dst_pp = jnp.concatenate([dst, jnp.zeros((pad,), jnp.int32)]).reshape(NW, EW)

    xp = jnp.zeros((NC * NPH, D), jnp.float32)
    xp = xp.at[:NH].set(x[:NH])
    xp = xp.at[NPH:NPH + NH].set(x[NH:])

    seg_src, seg_dst, lens, pcnt = _make_partition()(src_pp, dst_pp)
    seg_src = seg_src.reshape(NW, NC, SCAP, C)
    seg_dst = seg_dst.reshape(NW, NC, SCAP, C)
    pcT = jnp.zeros((NC * NPH, NW), jnp.float32)
    pcT = pcT.at[:NH].set(pcnt.T[:NH])
    pcT = pcT.at[NPH:NPH + NH].set(pcnt.T[NH:NC * NH])

    pa1 = _make_agg()(xp, seg_src, seg_dst, lens)
    h = _combine(pa1, pcT, xp, W1l.T, W1r.T, b1l.reshape(1, D), relu=True)
    pa2 = _make_agg()(h, seg_src, seg_dst, lens)
    outp = _combine(pa2, pcT, h, W2l.T, W2r.T, b2l.reshape(1, D), relu=False)
    return jnp.concatenate([outp[:NH], outp[NPH:NPH + NH]])


# speedup vs baseline: 2.0388x; 1.1989x over previous
import functools

import jax
import jax.numpy as jnp
from jax import lax
from jax.experimental import pallas as pl
from jax.experimental.pallas import tpu as pltpu
from jax.experimental.pallas import tpu_sc as plsc

N = 10000
D = 128
DH = D // 2
E = 320000

NC = 2
NS = 16
C = 128
ET = 20480
K = ET // C
EP = NS * ET
NP = 10240
RPT = NP // NS


NBUF = 4


def _sc_agg_body(with_cnt, *refs):
    if with_cnt:
        (xs_hbm, src_hbm, dst_hbm, acc_out, cnt_out,
         src_v, dst_v, rows, ones16, zeros16, acc_sh, cnt_sh,
         gsem, ssem, csem) = refs
    else:
        (xs_hbm, src_hbm, dst_hbm, acc_out,
         src_v, dst_v, rows, acc_sh, gsem, ssem) = refs
        csem = None

    c = lax.axis_index("c")
    s = lax.axis_index("s")

    pltpu.sync_copy(src_hbm.at[s], src_v)
    pltpu.sync_copy(dst_hbm.at[s], dst_v)

    zv = jnp.zeros((16,), jnp.float32)

    def zrow(r, carry):
        for q in range(DH // 16):
            rows[0, r, pl.ds(16 * q, 16)] = zv
        return carry

    lax.fori_loop(0, C, zrow, 0)
    for t in range(RPT // C):
        pltpu.sync_copy(rows.at[0], acc_sh.at[pl.ds(s * RPT + t * C, C)])

    if with_cnt:
        ov = jnp.full((16,), 1.0, jnp.float32)

        def orow(r, carry):
            ones16[r, pl.ds(0, 16)] = ov
            zeros16[r, pl.ds(0, 16)] = zv
            return carry

        lax.fori_loop(0, C, orow, 0)
        for t in range(RPT // C):
            pltpu.sync_copy(zeros16, cnt_sh.at[pl.ds(s * RPT + t * C, C)])

    plsc.subcore_barrier()

    xh = xs_hbm.at[c]
    for b in range(NBUF):
        pltpu.async_copy(xh.at[src_v.at[b]], rows.at[b], gsem.at[b])

    def step(t, carry):
        j0 = NBUF * t
        want_cnt = jnp.where(c == 0, j0 < K // 2, j0 >= K // 2)
        for b in range(NBUF):
            j = j0 + b
            pltpu.make_async_copy(xh.at[src_v.at[j]], rows.at[b], gsem.at[b]).wait()
            pltpu.async_copy(rows.at[b], acc_sh.at[dst_v.at[j]], ssem.at[b], add=True)
            if with_cnt:
                @pl.when(want_cnt)
                def _():
                    pltpu.async_copy(ones16, cnt_sh.at[dst_v.at[j]], csem,
                                     add=True)

        @pl.when(t < K // NBUF - 1)
        def _():
            for b in range(NBUF):
                j = j0 + b
                pltpu.make_async_copy(rows.at[b], acc_sh.at[dst_v.at[j]],
                                      ssem.at[b]).wait()
                pltpu.async_copy(xh.at[src_v.at[j + NBUF]], rows.at[b],
                                 gsem.at[b])
            if with_cnt:
                @pl.when(want_cnt)
                def _():
                    for b in range(NBUF):
                        pltpu.make_async_copy(ones16,
                                              cnt_sh.at[dst_v.at[j0 + b]],
                                              csem).wait()

        return carry

    lax.fori_loop(0, K // NBUF, step, 0)

    for b in range(NBUF):
        j = K - NBUF + b
        pltpu.make_async_copy(rows.at[b], acc_sh.at[dst_v.at[j]], ssem.at[b]).wait()
        if with_cnt:
            @pl.when(c != 0)
            def _():
                pltpu.make_async_copy(ones16, cnt_sh.at[dst_v.at[j]],
                                      csem).wait()

    plsc.subcore_barrier()

    pltpu.sync_copy(acc_sh.at[pl.ds(s * RPT, RPT)],
                    acc_out.at[c, pl.ds(s * RPT, RPT)])
    if with_cnt:
        pltpu.sync_copy(cnt_sh.at[pl.ds(s * RPT, RPT)],
                        cnt_out.at[c, pl.ds(s * RPT, RPT)])


def _make_sc_agg(with_cnt):
    mesh = plsc.VectorSubcoreMesh(core_axis_name="c", subcore_axis_name="s")
    out_type = [jax.ShapeDtypeStruct((NC, NP, DH), jnp.float32)]
    scratch = [
        pltpu.VMEM((K, C), jnp.int32),
        pltpu.VMEM((K, C), jnp.int32),
        pltpu.VMEM((NBUF, C, DH), jnp.float32),
    ]
    if with_cnt:
        out_type.append(jax.ShapeDtypeStruct((NC, NP, 16), jnp.float32))
        scratch += [
            pltpu.VMEM((C, 16), jnp.float32),
            pltpu.VMEM((C, 16), jnp.float32),
        ]
    scratch.append(pltpu.VMEM_SHARED((NP, DH), jnp.float32))
    if with_cnt:
        scratch.append(pltpu.VMEM_SHARED((NP, 16), jnp.float32))
    scratch += [pltpu.SemaphoreType.DMA((NBUF,)),
                pltpu.SemaphoreType.DMA((NBUF,))]
    if with_cnt:
        scratch.append(pltpu.SemaphoreType.DMA)

    return pl.kernel(
        functools.partial(_sc_agg_body, with_cnt),
        out_type=tuple(out_type),
        mesh=mesh,
        scratch_types=tuple(scratch),
        compiler_params=pltpu.CompilerParams(use_tc_tiling_on_sc=False),
    )


def _combine_body(relu, pa_ref, pc_ref, xin_ref, wl_ref, wr_ref, b_ref,
                  out_ref):
    acc = jnp.concatenate([pa_ref[0], pa_ref[1]], axis=1)
    xin = jnp.concatenate([xin_ref[0], xin_ref[1]], axis=1)
    cnt = pc_ref[0, :, 0:1] + pc_ref[1, :, 0:1]
    mean = acc * (1.0 / jnp.maximum(cnt, 1.0))
    y = (jnp.dot(mean, wl_ref[...], preferred_element_type=jnp.float32)
         + b_ref[...]
         + jnp.dot(xin, wr_ref[...], preferred_element_type=jnp.float32))
    if relu:
        h = jnp.maximum(y, 0.0)
        out_ref[0] = h[:, :DH]
        out_ref[1] = h[:, DH:]
    else:
        out_ref[...] = y


def _combine(pa, pc, xin, wlT, wrT, b2d, relu):
    R = 2000
    grid = (N // R,)
    if relu:
        out_shape = jax.ShapeDtypeStruct((NC, N, DH), jnp.float32)
        out_spec = pl.BlockSpec((NC, R, DH), lambda i: (0, i, 0))
    else:
        out_shape = jax.ShapeDtypeStruct((N, D), jnp.float32)
        out_spec = pl.BlockSpec((R, D), lambda i: (i, 0))
    return pl.pallas_call(
        functools.partial(_combine_body, relu),
        grid=grid,
        in_specs=[
            pl.BlockSpec((NC, R, DH), lambda i: (0, i, 0)),
            pl.BlockSpec((NC, R, 16), lambda i: (0, i, 0)),
            pl.BlockSpec((NC, R, DH), lambda i: (0, i, 0)),
            pl.BlockSpec((D, D), lambda i: (0, 0)),
            pl.BlockSpec((D, D), lambda i: (0, 0)),
            pl.BlockSpec((1, D), lambda i: (0, 0)),
        ],
        out_specs=out_spec,
        out_shape=out_shape,
    )(pa, pc, xin, wlT, wrT, b2d)


@jax.jit
def kernel(x, edge_index, W1l, b1l, W1r, W2l, b2l, W2r):
    src = edge_index[0]
    dst = edge_index[1]
    pad = EP - E
    src_p = jnp.concatenate([src, jnp.zeros((pad,), jnp.int32)]).reshape(NS, K, C)
    dst_p = jnp.concatenate([dst, jnp.full((pad,), NP - 1, jnp.int32)]).reshape(NS, K, C)
    xs = jnp.stack([x[:, :DH], x[:, DH:]])

    agg1 = _make_sc_agg(True)
    agg2 = _make_sc_agg(False)

    pa1, pc = agg1(xs, src_p, dst_p)
    hs = _combine(pa1, pc, xs, W1l.T, W1r.T, b1l.reshape(1, D), relu=True)
    pa2 = agg2(hs, src_p, dst_p)
    if isinstance(pa2, (tuple, list)):
        pa2 = pa2[0]
    out = _combine(pa2, pc, hs, W2l.T, W2r.T, b2l.reshape(1, D), relu=False)
    return out


# speedup vs baseline: 2.1224x; 1.0410x over previous
import functools

import jax
import jax.numpy as jnp
from jax import lax
from jax.experimental import pallas as pl
from jax.experimental.pallas import tpu as pltpu
from jax.experimental.pallas import tpu_sc as plsc

N = 10000
D = 128
DH = D // 2
E = 320000

NC = 2
NS = 16
C = 128
ET = 20480
K = ET // C
EP = NS * ET
NP = 10240
RPT = NP // NS


NBUF = 4


def _sc_agg_body(with_cnt, *refs):
    if with_cnt:
        (xs_hbm, src_hbm, dst_hbm, acc_out, cnt_out,
         src_v, dst_v, rows, ones16, zeros16, acc_sh, cnt_sh,
         gsem, ssem, csem) = refs
    else:
        (xs_hbm, src_hbm, dst_hbm, acc_out,
         src_v, dst_v, rows, acc_sh, gsem, ssem) = refs
        csem = None

    c = lax.axis_index("c")
    s = lax.axis_index("s")

    pltpu.sync_copy(src_hbm.at[s], src_v)
    pltpu.sync_copy(dst_hbm.at[s], dst_v)

    zv = jnp.zeros((16,), jnp.float32)

    def zrow(r, carry):
        for q in range(DH // 16):
            rows[0, r, pl.ds(16 * q, 16)] = zv
        return carry

    lax.fori_loop(0, C, zrow, 0)
    for t in range(RPT // C):
        pltpu.sync_copy(rows.at[0], acc_sh.at[pl.ds(s * RPT + t * C, C)])

    if with_cnt:
        ov = jnp.full((16,), 1.0, jnp.float32)

        def orow(r, carry):
            ones16[r, pl.ds(0, 16)] = ov
            zeros16[r, pl.ds(0, 16)] = zv
            return carry

        lax.fori_loop(0, C, orow, 0)
        for t in range(RPT // C):
            pltpu.sync_copy(zeros16, cnt_sh.at[pl.ds(s * RPT + t * C, C)])

    plsc.subcore_barrier()

    xh = xs_hbm.at[c]
    for b in range(NBUF):
        pltpu.async_copy(xh.at[src_v.at[b]], rows.at[b], gsem.at[b])

    def step(t, carry):
        j0 = NBUF * t
        want_cnt = jnp.where(c == 0, j0 < K // 2, j0 >= K // 2)
        for b in range(NBUF):
            j = j0 + b
            pltpu.make_async_copy(xh.at[src_v.at[j]], rows.at[b], gsem.at[b]).wait()
            pltpu.async_copy(rows.at[b], acc_sh.at[dst_v.at[j]], ssem.at[b], add=True)
            if with_cnt:
                @pl.when(want_cnt)
                def _():
                    pltpu.async_copy(ones16, cnt_sh.at[dst_v.at[j]], csem,
                                     add=True)

        @pl.when(t < K // NBUF - 1)
        def _():
            for b in range(NBUF):
                j = j0 + b
                pltpu.make_async_copy(rows.at[b], acc_sh.at[dst_v.at[j]],
                                      ssem.at[b]).wait()
                pltpu.async_copy(xh.at[src_v.at[j + NBUF]], rows.at[b],
                                 gsem.at[b])
            if with_cnt:
                @pl.when(want_cnt)
                def _():
                    for b in range(NBUF):
                        pltpu.make_async_copy(ones16,
                                              cnt_sh.at[dst_v.at[j0 + b]],
                                              csem).wait()

        return carry

    lax.fori_loop(0, K // NBUF, step, 0)

    for b in range(NBUF):
        j = K - NBUF + b
        pltpu.make_async_copy(rows.at[b], acc_sh.at[dst_v.at[j]], ssem.at[b]).wait()
        if with_cnt:
            @pl.when(c != 0)
            def _():
                pltpu.make_async_copy(ones16, cnt_sh.at[dst_v.at[j]],
                                      csem).wait()

    plsc.subcore_barrier()

    pltpu.sync_copy(acc_sh.at[pl.ds(s * RPT, RPT)],
                    acc_out.at[c, pl.ds(s * RPT, RPT)])
    if with_cnt:
        pltpu.sync_copy(cnt_sh.at[pl.ds(s * RPT, RPT)],
                        cnt_out.at[c, pl.ds(s * RPT, RPT)])


def _make_sc_agg(with_cnt):
    mesh = plsc.VectorSubcoreMesh(core_axis_name="c", subcore_axis_name="s")
    out_type = [jax.ShapeDtypeStruct((NC, NP, DH), jnp.float32)]
    scratch = [
        pltpu.VMEM((K, C), jnp.int32),
        pltpu.VMEM((K, C), jnp.int32),
        pltpu.VMEM((NBUF, C, DH), jnp.float32),
    ]
    if with_cnt:
        out_type.append(jax.ShapeDtypeStruct((NC, NP, 16), jnp.float32))
        scratch += [
            pltpu.VMEM((C, 16), jnp.float32),
            pltpu.VMEM((C, 16), jnp.float32),
        ]
    scratch.append(pltpu.VMEM_SHARED((NP, DH), jnp.float32))
    if with_cnt:
        scratch.append(pltpu.VMEM_SHARED((NP, 16), jnp.float32))
    scratch += [pltpu.SemaphoreType.DMA((NBUF,)),
                pltpu.SemaphoreType.DMA((NBUF,))]
    if with_cnt:
        scratch.append(pltpu.SemaphoreType.DMA)

    return pl.kernel(
        functools.partial(_sc_agg_body, with_cnt),
        out_type=tuple(out_type),
        mesh=mesh,
        scratch_types=tuple(scratch),
        compiler_params=pltpu.CompilerParams(use_tc_tiling_on_sc=False),
    )


def _combine_body(relu, pa_ref, pc_ref, xin_ref, wl_ref, wr_ref, b_ref,
                  out_ref):
    acc = jnp.concatenate([pa_ref[0], pa_ref[1]], axis=1)
    xin = jnp.concatenate([xin_ref[0], xin_ref[1]], axis=1)
    cnt = pc_ref[0, :, 0:1] + pc_ref[1, :, 0:1]
    mean = acc * (1.0 / jnp.maximum(cnt, 1.0))
    y = (jnp.dot(mean, wl_ref[...], preferred_element_type=jnp.float32)
         + b_ref[...]
         + jnp.dot(xin, wr_ref[...], preferred_element_type=jnp.float32))
    if relu:
        h = jnp.maximum(y, 0.0)
        out_ref[0] = h[:, :DH]
        out_ref[1] = h[:, DH:]
    else:
        out_ref[...] = y


def _combine(pa, pc, xin, wlT, wrT, b2d, relu):
    R = 5000
    grid = (N // R,)
    if relu:
        out_shape = jax.ShapeDtypeStruct((NC, N, DH), jnp.float32)
        out_spec = pl.BlockSpec((NC, R, DH), lambda i: (0, i, 0))
    else:
        out_shape = jax.ShapeDtypeStruct((N, D), jnp.float32)
        out_spec = pl.BlockSpec((R, D), lambda i: (i, 0))
    return pl.pallas_call(
        functools.partial(_combine_body, relu),
        grid=grid,
        in_specs=[
            pl.BlockSpec((NC, R, DH), lambda i: (0, i, 0)),
            pl.BlockSpec((NC, R, 16), lambda i: (0, i, 0)),
            pl.BlockSpec((NC, R, DH), lambda i: (0, i, 0)),
            pl.BlockSpec((D, D), lambda i: (0, 0)),
            pl.BlockSpec((D, D), lambda i: (0, 0)),
            pl.BlockSpec((1, D), lambda i: (0, 0)),
        ],
        out_specs=out_spec,
        out_shape=out_shape,
    )(pa, pc, xin, wlT, wrT, b2d)


@jax.jit
def kernel(x, edge_index, W1l, b1l, W1r, W2l, b2l, W2r):
    src = edge_index[0]
    dst = edge_index[1]
    pad = EP - E
    src_p = jnp.concatenate([src, jnp.zeros((pad,), jnp.int32)]).reshape(NS, K, C)
    dst_p = jnp.concatenate([dst, jnp.full((pad,), NP - 1, jnp.int32)]).reshape(NS, K, C)
    xs = jnp.stack([x[:, :DH], x[:, DH:]])

    agg1 = _make_sc_agg(True)
    agg2 = _make_sc_agg(False)

    pa1, pc = agg1(xs, src_p, dst_p)
    hs = _combine(pa1, pc, xs, W1l.T, W1r.T, b1l.reshape(1, D), relu=True)
    pa2 = agg2(hs, src_p, dst_p)
    if isinstance(pa2, (tuple, list)):
        pa2 = pa2[0]
    out = _combine(pa2, pc, hs, W2l.T, W2r.T, b2l.reshape(1, D), relu=False)
    return out
